# baseline (speedup 1.0000x reference)
import functools

import jax
import jax.numpy as jnp
from jax.experimental import pallas as pl
from jax.experimental.pallas import tpu as pltpu

ALPHA_ = 0.1
BM = 512
BK = 1024


def _prop_kernel(adj_ref, x_ref, h_ref, o_ref, acc_ref, *, nk):
    k = pl.program_id(0)
    i = pl.program_id(1)
    part = jnp.dot(adj_ref[...], x_ref[...], preferred_element_type=jnp.float32)
    sl = pl.ds(i * BM, BM)

    @pl.when(k == 0)
    def _init():
        acc_ref[sl, :] = part

    @pl.when(jnp.logical_and(k > 0, k < nk - 1))
    def _accum():
        acc_ref[sl, :] += part

    @pl.when(k == nk - 1)
    def _epilogue():
        o_ref[...] = (1.0 - ALPHA_) * (acc_ref[sl, :] + part) + ALPHA_ * h_ref[...]


@jax.jit
def kernel(x, adj, h):
    n, d = x.shape
    nm = n // BM
    nk = n // BK
    last = nk - 1
    return pl.pallas_call(
        functools.partial(_prop_kernel, nk=nk),
        grid=(nk, nm),
        in_specs=[
            pl.BlockSpec((BM, BK), lambda k, i: (i, k)),
            pl.BlockSpec((BK, d), lambda k, i: (k, 0)),
            pl.BlockSpec((BM, d), lambda k, i: (jnp.where(k == last, i, 0), 0)),
        ],
        out_specs=pl.BlockSpec((BM, d), lambda k, i: (jnp.where(k == last, i, 0), 0)),
        out_shape=jax.ShapeDtypeStruct((n, d), jnp.float32),
        scratch_shapes=[pltpu.VMEM((n, d), jnp.float32)],
        compiler_params=pltpu.CompilerParams(
            dimension_semantics=("arbitrary", "arbitrary"),
        ),
    )(adj, x, h)


# speedup vs baseline: 1.4290x; 1.4290x over previous
import jax
import jax.numpy as jnp
from jax.experimental import pallas as pl
from jax.experimental.pallas import tpu as pltpu

ALPHA_ = 0.1
BM = 512


def _prop_kernel(adj_top_ref, adj_bot_ref, x_ref, h_ref, o_ref):
    xv = x_ref[...]
    o_ref[0] = (1.0 - ALPHA_) * jnp.dot(
        adj_top_ref[...], xv, preferred_element_type=jnp.float32
    ) + ALPHA_ * h_ref[0]
    o_ref[1] = (1.0 - ALPHA_) * jnp.dot(
        adj_bot_ref[...], xv, preferred_element_type=jnp.float32
    ) + ALPHA_ * h_ref[1]


@jax.jit
def kernel(x, adj, h):
    n, d = x.shape
    half = n // 2
    nm = half // BM
    h2 = h.reshape(2, half, d)
    out = pl.pallas_call(
        _prop_kernel,
        grid=(nm,),
        in_specs=[
            pl.BlockSpec((BM, n), lambda i: (i, 0)),
            pl.BlockSpec((BM, n), lambda i, _nm=nm: (i + _nm, 0)),
            pl.BlockSpec((n, d), lambda i: (0, 0)),
            pl.BlockSpec((2, BM, d), lambda i: (0, i, 0)),
        ],
        out_specs=pl.BlockSpec((2, BM, d), lambda i: (0, i, 0)),
        out_shape=jax.ShapeDtypeStruct((2, half, d), jnp.float32),
        compiler_params=pltpu.CompilerParams(
            dimension_semantics=("arbitrary",),
        ),
    )(adj, adj, x, h2)
    return out.reshape(n, d)
